# baseline (speedup 1.0000x reference)
import functools

import jax
import jax.numpy as jnp
from jax import lax
from jax.experimental import pallas as pl
from jax.experimental.pallas import tpu as pltpu
from jax.experimental.pallas import tpu_sc as plsc

_PAD = 1


def _positions_body(x_ref, out_ref):
    x = x_ref[...]
    m = (x != _PAD).astype(jnp.int32)
    s = m
    shift = 1
    n = x.shape[1]
    while shift < n:
        z = jnp.zeros((x.shape[0], shift), jnp.int32)
        s = s + jnp.concatenate([z, s[:, :-shift]], axis=1)
        shift *= 2
    out_ref[...] = s * m + _PAD


def _compute_positions(inp):
    return pl.pallas_call(
        _positions_body,
        out_shape=jax.ShapeDtypeStruct(inp.shape, jnp.int32),
    )(inp)


def _make_gather(rows, d, n_workers, chunk, nbuf):
    b_per_w = rows // n_workers
    n_chunks = b_per_w // chunk
    n_outer = n_chunks // nbuf
    mesh = plsc.VectorSubcoreMesh(core_axis_name="c", subcore_axis_name="s")

    @functools.partial(
        pl.kernel,
        mesh=mesh,
        out_type=jax.ShapeDtypeStruct((rows, d), jnp.float32),
        scratch_types=[
            pltpu.VMEM((b_per_w,), jnp.int32),
        ]
        + [pltpu.VMEM((chunk, d), jnp.float32) for _ in range(nbuf)]
        + [pltpu.SemaphoreType.DMA for _ in range(2 * nbuf)],
    )
    def gather_kernel(table_hbm, idx_hbm, out_hbm, idx_v, *refs):
        bufs = refs[:nbuf]
        gsems = refs[nbuf : 2 * nbuf]
        wsems = refs[2 * nbuf :]
        num_cores = 2
        wid = lax.axis_index("s") * num_cores + lax.axis_index("c")
        base = wid * b_per_w
        pltpu.sync_copy(idx_hbm.at[pl.ds(base, b_per_w)], idx_v)

        def start_gather(off, b):
            pltpu.async_copy(
                table_hbm.at[idx_v.at[pl.ds(off, chunk)]], bufs[b], gsems[b]
            )

        def wait_gather(off, b):
            pltpu.make_async_copy(
                table_hbm.at[idx_v.at[pl.ds(off, chunk)]], bufs[b], gsems[b]
            ).wait()

        def start_write(off, b):
            pltpu.async_copy(
                bufs[b], out_hbm.at[pl.ds(base + off, chunk)], wsems[b]
            )

        def wait_write(off, b):
            pltpu.make_async_copy(
                bufs[b], out_hbm.at[pl.ds(base + off, chunk)], wsems[b]
            ).wait()

        for b in range(nbuf):
            start_gather(b * chunk, b)

        def outer(o, carry):
            for b in range(nbuf):
                off = (o * nbuf + b) * chunk
                wait_gather(off, b)
                start_write(off, b)
                wait_write(off, b)
                start_gather(off + nbuf * chunk, b)
            return carry

        lax.fori_loop(0, n_outer - 1, outer, 0)

        tail = (n_outer - 1) * nbuf * chunk
        for b in range(nbuf):
            off = tail + b * chunk
            wait_gather(off, b)
            start_write(off, b)
        for b in range(nbuf):
            wait_write(tail + b * chunk, b)

    return gather_kernel


def _make_bcast_gather(batch, seq, d, n_workers, chunk):
    s_per_w = seq // n_workers
    n_chunks = s_per_w // chunk
    nbuf = 2
    mesh = plsc.VectorSubcoreMesh(core_axis_name="c", subcore_axis_name="s")

    @functools.partial(
        pl.kernel,
        mesh=mesh,
        out_type=jax.ShapeDtypeStruct((batch * seq, d), jnp.float32),
        scratch_types=[
            pltpu.VMEM((batch, s_per_w), jnp.int32),
            pltpu.VMEM((s_per_w,), jnp.int32),
        ]
        + [pltpu.VMEM((chunk, d), jnp.float32) for _ in range(nbuf)]
        + [pltpu.SemaphoreType.DMA for _ in range(2 * nbuf + 1)],
    )
    def bcast_kernel(table_hbm, pos_hbm, out_hbm, pos_v, idx_v, *refs):
        bufs = refs[:nbuf]
        bufd = refs[0]
        gsems = refs[nbuf : 2 * nbuf]
        wsems = refs[2 * nbuf : 3 * nbuf]
        dsem = refs[3 * nbuf]
        num_cores = 2
        wid = lax.axis_index("s") * num_cores + lax.axis_index("c")
        s0 = wid * s_per_w
        for b in range(batch):
            pltpu.sync_copy(
                pos_hbm.at[pl.ds(b * seq + s0, s_per_w)], pos_v.at[b]
            )
        for i in range(s_per_w // 16):
            idx_v[pl.ds(i * 16, 16)] = lax.iota(jnp.int32, 16) + s0 + i * 16 + 2

        def start_gather(c, i):
            pltpu.async_copy(
                table_hbm.at[idx_v.at[pl.ds(c * chunk, chunk)]], bufs[i], gsems[i]
            )

        def wait_gather(c, i):
            pltpu.make_async_copy(
                table_hbm.at[idx_v.at[pl.ds(c * chunk, chunk)]], bufs[i], gsems[i]
            ).wait()

        def start_writes(c, i):
            for b in range(batch):
                pltpu.async_copy(
                    bufs[i],
                    out_hbm.at[pl.ds(b * seq + s0 + c * chunk, chunk)],
                    wsems[i],
                )

        def wait_writes(c, i):
            for b in range(batch):
                pltpu.make_async_copy(
                    bufs[i],
                    out_hbm.at[pl.ds(b * seq + s0 + c * chunk, chunk)],
                    wsems[i],
                ).wait()

        for c in range(min(nbuf, n_chunks)):
            start_gather(c, c % nbuf)
        for c in range(n_chunks):
            wait_gather(c, c % nbuf)
            p = c - (nbuf - 1)
            if p >= 0 and p + nbuf < n_chunks:
                wait_writes(p, p % nbuf)
                start_gather(p + nbuf, p % nbuf)
            start_writes(c, c % nbuf)
        for c in range(max(0, n_chunks - (nbuf - 1) - 1), n_chunks):
            wait_writes(c, c % nbuf)

        for c in range(n_chunks):
            for b in range(batch):
                tail = pos_v[b, pl.ds(c * chunk + chunk - 16, 16)]
                dirty = tail[15] != (s0 + c * chunk + chunk + 1)

                @pl.when(dirty)
                def _():
                    pltpu.async_copy(
                        table_hbm.at[pos_v.at[b, pl.ds(c * chunk, chunk)]],
                        bufd,
                        dsem,
                    ).wait()
                    pltpu.sync_copy(
                        bufd, out_hbm.at[pl.ds(b * seq + s0 + c * chunk, chunk)]
                    )

    return bcast_kernel


def kernel(input, weights):
    b, s = input.shape
    d = weights.shape[1]
    positions = _compute_positions(input)
    flat = positions.reshape(-1)
    gather = _make_bcast_gather(b, s, d, 32, 32)
    out = gather(weights, flat)
    return out.reshape(b, s, d)


# speedup vs baseline: 1.0034x; 1.0034x over previous
import functools

import jax
import jax.numpy as jnp
from jax import lax
from jax.experimental import pallas as pl
from jax.experimental.pallas import tpu as pltpu
from jax.experimental.pallas import tpu_sc as plsc

_PAD = 1


def _positions_body(x_ref, out_ref):
    x = x_ref[...]
    m = (x != _PAD).astype(jnp.int32)
    s = m
    shift = 1
    n = x.shape[1]
    while shift < n:
        z = jnp.zeros((x.shape[0], shift), jnp.int32)
        s = s + jnp.concatenate([z, s[:, :-shift]], axis=1)
        shift *= 2
    out_ref[...] = s * m + _PAD


def _compute_positions(inp):
    return pl.pallas_call(
        _positions_body,
        out_shape=jax.ShapeDtypeStruct(inp.shape, jnp.int32),
    )(inp)


def _make_bcast_gather(batch, seq, d, n_workers, chunk):
    s_per_w = seq // n_workers
    n_chunks = s_per_w // chunk
    nbuf = 2
    mesh = plsc.VectorSubcoreMesh(core_axis_name="c", subcore_axis_name="s")

    @functools.partial(
        pl.kernel,
        mesh=mesh,
        out_type=jax.ShapeDtypeStruct((batch * seq, d), jnp.float32),
        scratch_types=[
            pltpu.VMEM((batch, s_per_w), jnp.int32),
            pltpu.VMEM((s_per_w,), jnp.int32),
        ]
        + [pltpu.VMEM((chunk, d), jnp.float32) for _ in range(nbuf)]
        + [pltpu.SemaphoreType.DMA for _ in range(2 * nbuf + 1)],
    )
    def bcast_kernel(table_hbm, pos_hbm, out_hbm, pos_v, idx_v, *refs):
        bufs = refs[:nbuf]
        bufd = refs[0]
        gsems = refs[nbuf : 2 * nbuf]
        wsems = refs[2 * nbuf : 3 * nbuf]
        dsem = refs[3 * nbuf]
        num_cores = 2
        wid = lax.axis_index("s") * num_cores + lax.axis_index("c")
        s0 = wid * s_per_w
        for b in range(batch):
            pltpu.sync_copy(
                pos_hbm.at[pl.ds(b * seq + s0, s_per_w)], pos_v.at[b]
            )
        for i in range(s_per_w // 16):
            idx_v[pl.ds(i * 16, 16)] = lax.iota(jnp.int32, 16) + s0 + i * 16 + 2

        def start_gather(c, i):
            pltpu.async_copy(
                table_hbm.at[idx_v.at[pl.ds(c * chunk, chunk)]], bufs[i], gsems[i]
            )

        def wait_gather(c, i):
            pltpu.make_async_copy(
                table_hbm.at[idx_v.at[pl.ds(c * chunk, chunk)]], bufs[i], gsems[i]
            ).wait()

        def start_writes(c, i):
            for b in range(batch):
                pltpu.async_copy(
                    bufs[i],
                    out_hbm.at[pl.ds(b * seq + s0 + c * chunk, chunk)],
                    wsems[i],
                )

        def wait_writes(c, i):
            for b in range(batch):
                pltpu.make_async_copy(
                    bufs[i],
                    out_hbm.at[pl.ds(b * seq + s0 + c * chunk, chunk)],
                    wsems[i],
                ).wait()

        for c in range(min(nbuf, n_chunks)):
            start_gather(c, c % nbuf)
        for c in range(n_chunks):
            wait_gather(c, c % nbuf)
            p = c - (nbuf - 1)
            if p >= 0 and p + nbuf < n_chunks:
                wait_writes(p, p % nbuf)
                start_gather(p + nbuf, p % nbuf)
            start_writes(c, c % nbuf)
        for c in range(max(0, n_chunks - (nbuf - 1) - 1), n_chunks):
            wait_writes(c, c % nbuf)

        for c in range(n_chunks):
            for b in range(batch):
                tail = pos_v[b, pl.ds(c * chunk + chunk - 16, 16)]
                dirty = tail[15] != (s0 + c * chunk + chunk + 1)

                @pl.when(dirty)
                def _():
                    pltpu.async_copy(
                        table_hbm.at[pos_v.at[b, pl.ds(c * chunk, chunk)]],
                        bufd,
                        dsem,
                    ).wait()
                    pltpu.sync_copy(
                        bufd, out_hbm.at[pl.ds(b * seq + s0 + c * chunk, chunk)]
                    )

    return bcast_kernel


def kernel(input, weights):
    b, s = input.shape
    d = weights.shape[1]
    positions = _compute_positions(input)
    flat = positions.reshape(-1)
    gather = _make_bcast_gather(b, s, d, 32, 32)
    out = gather(weights, flat)
    return out.reshape(b, s, d)


# speedup vs baseline: 1.0140x; 1.0106x over previous
import functools

import jax
import jax.numpy as jnp
from jax import lax
from jax.experimental import pallas as pl
from jax.experimental.pallas import tpu as pltpu
from jax.experimental.pallas import tpu_sc as plsc

_PAD = 1


def _positions_body(x_ref, out_ref):
    x = x_ref[...]
    m = (x != _PAD).astype(jnp.int32)
    s = m
    shift = 1
    n = x.shape[1]
    while shift < n:
        z = jnp.zeros((x.shape[0], shift), jnp.int32)
        s = s + jnp.concatenate([z, s[:, :-shift]], axis=1)
        shift *= 2
    out_ref[...] = s * m + _PAD


def _compute_positions(inp):
    return pl.pallas_call(
        _positions_body,
        out_shape=jax.ShapeDtypeStruct(inp.shape, jnp.int32),
    )(inp)


def _make_bcast_gather(batch, seq, d, n_workers, chunk):
    s_per_w = seq // n_workers
    n_chunks = s_per_w // chunk
    nbuf = 2
    mesh = plsc.VectorSubcoreMesh(core_axis_name="c", subcore_axis_name="s")

    @functools.partial(
        pl.kernel,
        mesh=mesh,
        out_type=jax.ShapeDtypeStruct((batch * seq, d), jnp.float32),
        scratch_types=[
            pltpu.VMEM((batch, s_per_w), jnp.int32),
            pltpu.VMEM((s_per_w,), jnp.int32),
        ]
        + [pltpu.VMEM((chunk, d), jnp.float32) for _ in range(nbuf)]
        + [pltpu.SemaphoreType.DMA for _ in range(2 * nbuf + 1)],
    )
    def bcast_kernel(table_hbm, pos_hbm, out_hbm, pos_v, idx_v, *refs):
        bufs = refs[:nbuf]
        bufd = refs[0]
        gsems = refs[nbuf : 2 * nbuf]
        wsems = refs[2 * nbuf : 3 * nbuf]
        dsem = refs[3 * nbuf]
        num_cores = 2
        wid = lax.axis_index("s") * num_cores + lax.axis_index("c")
        s0 = wid * s_per_w
        for i in range(s_per_w // 16):
            idx_v[pl.ds(i * 16, 16)] = lax.iota(jnp.int32, 16) + s0 + i * 16 + 2

        def start_gather(c, i):
            pltpu.async_copy(
                table_hbm.at[idx_v.at[pl.ds(c * chunk, chunk)]], bufs[i], gsems[i]
            )

        def wait_gather(c, i):
            pltpu.make_async_copy(
                table_hbm.at[idx_v.at[pl.ds(c * chunk, chunk)]], bufs[i], gsems[i]
            ).wait()

        def start_writes(c, i):
            for b in range(batch):
                pltpu.async_copy(
                    bufs[i],
                    out_hbm.at[pl.ds(b * seq + s0 + c * chunk, chunk)],
                    wsems[i],
                )

        def wait_writes(c, i):
            for b in range(batch):
                pltpu.make_async_copy(
                    bufs[i],
                    out_hbm.at[pl.ds(b * seq + s0 + c * chunk, chunk)],
                    wsems[i],
                ).wait()

        for c in range(min(nbuf, n_chunks)):
            start_gather(c, c % nbuf)
        for b in range(batch):
            pltpu.sync_copy(
                pos_hbm.at[pl.ds(b * seq + s0, s_per_w)], pos_v.at[b]
            )
        for c in range(n_chunks):
            wait_gather(c, c % nbuf)
            p = c - (nbuf - 1)
            if p >= 0 and p + nbuf < n_chunks:
                wait_writes(p, p % nbuf)
                start_gather(p + nbuf, p % nbuf)
            start_writes(c, c % nbuf)
        for c in range(max(0, n_chunks - (nbuf - 1) - 1), n_chunks):
            wait_writes(c, c % nbuf)

        for c in range(n_chunks):
            for b in range(batch):
                tail = pos_v[b, pl.ds(c * chunk + chunk - 16, 16)]
                dirty = tail[15] != (s0 + c * chunk + chunk + 1)

                @pl.when(dirty)
                def _():
                    pltpu.async_copy(
                        table_hbm.at[pos_v.at[b, pl.ds(c * chunk, chunk)]],
                        bufd,
                        dsem,
                    ).wait()
                    pltpu.sync_copy(
                        bufd, out_hbm.at[pl.ds(b * seq + s0 + c * chunk, chunk)]
                    )

    return bcast_kernel


def kernel(input, weights):
    b, s = input.shape
    d = weights.shape[1]
    positions = _compute_positions(input)
    flat = positions.reshape(-1)
    gather = _make_bcast_gather(b, s, d, 32, 32)
    out = gather(weights, flat)
    return out.reshape(b, s, d)
